# baseline (speedup 1.0000x reference)
import functools

import jax
import jax.numpy as jnp
from jax import lax
from jax.experimental import pallas as pl
from jax.experimental.pallas import tpu as pltpu
from jax.experimental.pallas import tpu_sc as plsc

N = 10000
E = 160000
D = 256
HALF = 128
AUGW = 144
NC = 2
NS = 16
LANES = 16
B = 64
NB_PT = 160
E_PAD = NB_PT * NS * B
ROWS_PAD = E_PAD // B
NBUF = 4
NPT = N // NS
BLK = 1000
NBLK = N // BLK


def _make_sc_aggregate(width):
  mesh = plsc.VectorSubcoreMesh(
      core_axis_name="c", subcore_axis_name="s", num_cores=NC, num_subcores=NS)

  @functools.partial(
      pl.kernel,
      out_type=jax.ShapeDtypeStruct((NC * N, width), jnp.float32),
      mesh=mesh,
      scratch_types=[
          pltpu.VMEM_SHARED((N, width), jnp.float32),
      ] + [pltpu.VMEM((B, width), jnp.float32) for _ in range(NBUF)]
        + [pltpu.VMEM((3, B), jnp.int32) for _ in range(NBUF)]
        + [pltpu.VMEM((B,), jnp.float32) for _ in range(NBUF)]
        + [pltpu.SemaphoreType.DMA for _ in range(3 * NBUF)],
      compiler_params=pltpu.CompilerParams(use_tc_tiling_on_sc=False),
  )
  def agg(table_hbm, packed_hbm, w_hbm, zeros_hbm, out_hbm, acc, *scr):
    rows = scr[0:NBUF]
    idxb = scr[NBUF:2 * NBUF]
    wb = scr[2 * NBUF:3 * NBUF]
    gsem = scr[3 * NBUF:4 * NBUF]
    ssem = scr[4 * NBUF:5 * NBUF]
    isem = scr[5 * NBUF:6 * NBUF]
    c = lax.axis_index("c")
    s = lax.axis_index("s")
    base_r = s * NB_PT

    pltpu.sync_copy(zeros_hbm, acc.at[pl.ds(s * NPT, NPT)])
    plsc.subcore_barrier()

    def idx_start(j, b):
      r = base_r + b
      pltpu.async_copy(packed_hbm.at[r], idxb[j], isem[j])
      pltpu.async_copy(w_hbm.at[r], wb[j], isem[j])

    def idx_wait(j, b):
      r = base_r + b
      pltpu.make_async_copy(packed_hbm.at[r], idxb[j], isem[j]).wait()
      pltpu.make_async_copy(w_hbm.at[r], wb[j], isem[j]).wait()

    def gather_start(j):
      pltpu.async_copy(table_hbm.at[idxb[j].at[c]], rows[j], gsem[j])

    def gather_wait(j):
      pltpu.make_async_copy(table_hbm.at[idxb[j].at[c]], rows[j],
                            gsem[j]).wait()

    def scatter_start(j):
      pltpu.async_copy(rows[j], acc.at[idxb[j].at[2]], ssem[j], add=True)

    def scatter_wait(j):
      pltpu.make_async_copy(rows[j], acc.at[idxb[j].at[2]], ssem[j]).wait()

    def scale(j):
      rj = rows[j]
      wj = wb[j]

      def mul_chunk(kb, carry):
        kbase = kb * LANES
        wk_vec = wj[pl.ds(kbase, LANES)]
        for l in range(LANES):
          wk = wk_vec[l]
          for q in range(width // LANES):
            sl = pl.ds(q * LANES, LANES)
            rj[kbase + l, sl] = rj[kbase + l, sl] * wk
        return carry

      lax.fori_loop(0, B // LANES, mul_chunk, 0)

    idx_start(0, 0)
    idx_wait(0, 0)
    gather_start(0)
    idx_start(1, 1)

    def body(i, carry):
      for j in range(NBUF):
        bl = NBUF * i + j
        gather_wait(j)
        scale(j)
        scatter_start(j)
        p2 = (j + 2) % NBUF
        bn2 = bl + 2

        @pl.when(bn2 < NB_PT)
        def _():
          @pl.when(bn2 - NBUF >= 0)
          def _():
            scatter_wait(p2)
          idx_start(p2, bn2)

        p1 = (j + 1) % NBUF
        bn1 = bl + 1

        @pl.when(bn1 < NB_PT)
        def _():
          idx_wait(p1, bn1)
          gather_start(p1)

      return carry

    lax.fori_loop(0, NB_PT // NBUF, body, 0)
    for j in range(NBUF):
      scatter_wait(j)
    plsc.subcore_barrier()
    pltpu.sync_copy(acc.at[pl.ds(s * NPT, NPT)],
                    out_hbm.at[pl.ds(c * N + s * NPT, NPT)])

  return agg


_sc_agg_aug = _make_sc_aggregate(AUGW)
_sc_agg_half = _make_sc_aggregate(HALF)


def _tc_layer1(a1, w1, b1):
  def body(aa_ref, ab_ref, w1a_ref, w1b_ref, b1_ref, out_ref):
    aa = aa_ref[...]
    ab = ab_ref[...]
    deg = aa[:, HALF:HALF + 1]
    scale = jnp.where(deg > 0.0, 1.0 / deg, 0.0)
    xa = aa[:, :HALF] * scale
    xb = ab[:, :HALF] * scale
    h = (jnp.dot(xa, w1a_ref[...], preferred_element_type=jnp.float32)
         + jnp.dot(xb, w1b_ref[...], preferred_element_type=jnp.float32)
         + b1_ref[...])
    out_ref[...] = jnp.maximum(h, 0.0)

  return pl.pallas_call(
      body,
      grid=(2, NBLK),
      in_specs=[
          pl.BlockSpec((BLK, AUGW), lambda j, i: (i, 0)),
          pl.BlockSpec((BLK, AUGW), lambda j, i: (i + NBLK, 0)),
          pl.BlockSpec((HALF, HALF), lambda j, i: (0, j)),
          pl.BlockSpec((HALF, HALF), lambda j, i: (1, j)),
          pl.BlockSpec((1, HALF), lambda j, i: (0, j)),
      ],
      out_specs=pl.BlockSpec((BLK, HALF), lambda j, i: (j * NBLK + i, 0)),
      out_shape=jax.ShapeDtypeStruct((2 * N, HALF), jnp.float32),
      compiler_params=pltpu.CompilerParams(
          dimension_semantics=("parallel", "parallel")),
  )(a1, a1, w1, w1, b1.reshape(1, D))


def _tc_layer2(a2, a1, w2, b2, wd, bd, wc, bc):
  def body(a2a_ref, a2b_ref, dega_ref, w2a_ref, w2b_ref, b2_ref,
           wd_ref, bd_ref, wc_ref, bc_ref, out_ref, acc_ref):
    i = pl.program_id(0)

    @pl.when(i == 0)
    def _():
      acc_ref[...] = jnp.zeros_like(acc_ref)

    deg = dega_ref[...][:, HALF:HALF + 1]
    scale = jnp.where(deg > 0.0, 1.0 / deg, 0.0)
    xa = a2a_ref[...] * scale
    xb = a2b_ref[...] * scale
    h2 = (jnp.dot(xa, w2a_ref[...], preferred_element_type=jnp.float32)
          + jnp.dot(xb, w2b_ref[...], preferred_element_type=jnp.float32)
          + b2_ref[...])
    h2 = jnp.maximum(h2, 0.0)
    acc_ref[...] += jnp.sum(h2, axis=0, keepdims=True)

    @pl.when(i == NBLK - 1)
    def _():
      hg = acc_ref[...] * (1.0 / N)
      o1 = jnp.maximum(
          jnp.dot(hg, wd_ref[...], preferred_element_type=jnp.float32)
          + bd_ref[...], 0.0)
      out_ref[...] = (
          jnp.dot(o1, wc_ref[...], preferred_element_type=jnp.float32)
          + bc_ref[...])

  return pl.pallas_call(
      body,
      grid=(NBLK,),
      in_specs=[
          pl.BlockSpec((BLK, HALF), lambda i: (i, 0)),
          pl.BlockSpec((BLK, HALF), lambda i: (i + NBLK, 0)),
          pl.BlockSpec((BLK, AUGW), lambda i: (i, 0)),
          pl.BlockSpec((HALF, D), lambda i: (0, 0)),
          pl.BlockSpec((HALF, D), lambda i: (1, 0)),
          pl.BlockSpec((1, D), lambda i: (0, 0)),
          pl.BlockSpec((D, HALF), lambda i: (0, 0)),
          pl.BlockSpec((1, HALF), lambda i: (0, 0)),
          pl.BlockSpec((HALF, 10), lambda i: (0, 0)),
          pl.BlockSpec((1, 10), lambda i: (0, 0)),
      ],
      out_specs=pl.BlockSpec((1, 10), lambda i: (0, 0)),
      out_shape=jax.ShapeDtypeStruct((1, 10), jnp.float32),
      scratch_shapes=[pltpu.VMEM((1, D), jnp.float32)],
      compiler_params=pltpu.CompilerParams(
          dimension_semantics=("arbitrary",)),
  )(a2, a2, a1, w2, w2, b2.reshape(1, D), wd, bd.reshape(1, HALF),
    wc, bc.reshape(1, 10))


def kernel(in_feat, edge_weights, W1, b1, W2, b2, Wd, bd, Wc, bc, edge_index):
  npad = E_PAD - E
  src = jnp.concatenate([edge_index[0], jnp.zeros((npad,), jnp.int32)])
  dst = jnp.concatenate([edge_index[1], jnp.zeros((npad,), jnp.int32)])
  w = jnp.concatenate([edge_weights, jnp.zeros((npad,), jnp.float32)])
  packed = jnp.stack([src, src + N, dst], axis=0)
  packed = packed.reshape(3, ROWS_PAD, B).transpose(1, 0, 2)
  w_rows = w.reshape(ROWS_PAD, B)

  ones = jnp.ones((N, 1), jnp.float32)
  pad = jnp.zeros((N, AUGW - HALF - 1), jnp.float32)
  table1 = jnp.concatenate([
      jnp.concatenate([in_feat[:, :HALF], ones, pad], axis=1),
      jnp.concatenate([in_feat[:, HALF:], ones, pad], axis=1),
  ], axis=0)

  zeros_aug = jnp.zeros((NPT, AUGW), jnp.float32)
  zeros_half = jnp.zeros((NPT, HALF), jnp.float32)

  a1 = _sc_agg_aug(table1, packed, w_rows, zeros_aug)
  h = _tc_layer1(a1, W1, b1)
  a2 = _sc_agg_half(h, packed, w_rows, zeros_half)
  return _tc_layer2(a2, a1, W2, b2, Wd, bd, Wc, bc)
